# Initial kernel scaffold; baseline (speedup 1.0000x reference)
import jax
import jax.numpy as jnp
from jax.experimental import pallas as pl


def kernel(x):
    raise NotImplementedError("write your pallas kernel here")


# speedup vs baseline: 1.3962x; 1.3962x over previous
import jax
import jax.numpy as jnp
from jax import lax
from jax.experimental import pallas as pl

_K = 8
_N = 32768
_ROWS = 128
_BLOCK_ROWS = 8


def _softmax_topk_kernel(x_ref, vals_ref, idx_ref):
    x = x_ref[...]
    r, n = x.shape
    iota = lax.broadcasted_iota(jnp.int32, (r, n), 1)
    m = jnp.max(x, axis=1, keepdims=True)
    s = jnp.sum(jnp.exp(x - m), axis=1, keepdims=True)
    work = x
    vals = []
    idxs = []
    for _ in range(_K):
        cur = jnp.max(work, axis=1, keepdims=True)
        hit = work == cur
        ind = jnp.min(jnp.where(hit, iota, n), axis=1, keepdims=True)
        vals.append(cur)
        idxs.append(ind)
        work = jnp.where(iota == ind, -jnp.inf, work)
    v = jnp.concatenate(vals, axis=1)
    i = jnp.concatenate(idxs, axis=1)
    vals_ref[...] = jnp.exp(v - m) / s
    idx_ref[...] = i


def kernel(x):
    grid = (_ROWS // _BLOCK_ROWS,)
    vals, idx = pl.pallas_call(
        _softmax_topk_kernel,
        grid=grid,
        in_specs=[pl.BlockSpec((_BLOCK_ROWS, _N), lambda i: (i, 0))],
        out_specs=[
            pl.BlockSpec((_BLOCK_ROWS, _K), lambda i: (i, 0)),
            pl.BlockSpec((_BLOCK_ROWS, _K), lambda i: (i, 0)),
        ],
        out_shape=[
            jax.ShapeDtypeStruct((_ROWS, _K), jnp.float32),
            jax.ShapeDtypeStruct((_ROWS, _K), jnp.int32),
        ],
    )(x)
    return vals, idx
